# baseline (speedup 1.0000x reference)
import functools

import jax
import jax.numpy as jnp
from jax import lax
from jax.experimental import pallas as pl
from jax.experimental.pallas import tpu as pltpu
from jax.experimental.pallas import tpu_sc as plsc


def _fused_body(nnzf, R, J,
                vals_ref, rtw_ref, rtb_ref, w1t_ref, b1_ref, w2_ref, b2_ref,
                vout_ref, att_ref, big_ref, acc_ref, sc_ref, of_ref):
    p = pl.program_id(0)
    r = pl.program_id(1)
    j = pl.program_id(2)

    @pl.when(p == 0)
    def _phase0():
        blk = vals_ref[0, 0]

        @pl.when((r == 0) & (j == 0))
        def _():
            acc_ref[...] = jnp.zeros_like(acc_ref)

        srow = jnp.sum(blk, axis=0, keepdims=True)
        for rr in range(R):
            @pl.when(r == rr)
            def _(rr=rr):
                acc_ref[rr:rr + 1, :] += srow

        t = r * J + j
        big_ref[pl.ds(t, 1)] = vals_ref[0]

    @pl.when(p == 1)
    def _phase1():
        @pl.when((r == 0) & (j == 0))
        def _():
            rows = [jnp.sum(acc_ref[rr:rr + 1, :], axis=1, keepdims=True)
                    for rr in range(R)]
            sums = jnp.concatenate(rows, axis=0)
            feats = rtw_ref[...] * sums + nnzf * rtb_ref[...]
            h = jnp.sum(feats * w1t_ref[...], axis=0, keepdims=True) + b1_ref[...]
            h = jnp.maximum(h, 0.0)
            logits = (jnp.sum(h * w2_ref[...], axis=1, keepdims=True)
                      + b2_ref[...])
            m = jnp.max(logits, axis=0, keepdims=True)
            e = jnp.exp(logits - m)
            att = e / jnp.sum(e, axis=0, keepdims=True)
            att_ref[...] = att
            sc_ref[...] = jnp.broadcast_to(att * rtw_ref[...], sc_ref.shape)
            of_ref[...] = jnp.broadcast_to(att * rtb_ref[...], of_ref.shape)

        t = r * J + j
        blk = big_ref[pl.ds(t, 1)][0]
        for rr in range(R):
            @pl.when(r == rr)
            def _(rr=rr):
                vout_ref[0, 0] = (blk * sc_ref[rr:rr + 1, 0:1]
                                  + of_ref[rr:rr + 1, 0:1])


def _idx_copy_body(R, NNZ, idx_ref, out_ref, sem):
    c = lax.axis_index("c")
    s = lax.axis_index("s")
    wid = s * 2 + c
    row = wid // 4
    q = wid % 4
    L = NNZ // 4
    for rr in range(R):
        for ii in range(2):
            @pl.when(row == rr * 2 + ii)
            def _(rr=rr, ii=ii):
                cp = pltpu.make_async_copy(
                    idx_ref.at[rr, ii, pl.ds(q * L, L)],
                    out_ref.at[ii, rr, pl.ds(q * L, L)],
                    sem)
                cp.start()
                cp.wait()


def _make_idx_copy(R, NNZ):
    mesh = plsc.VectorSubcoreMesh(core_axis_name="c", subcore_axis_name="s")
    return pl.kernel(
        functools.partial(_idx_copy_body, R, NNZ),
        out_type=jax.ShapeDtypeStruct((2, R, NNZ), jnp.int32),
        mesh=mesh,
        scratch_types=[pltpu.SemaphoreType.DMA],
    )


def kernel(A_indices, A_values, rt_w, rt_b, W1, b1, W2, b2):
    R, NNZ = A_values.shape
    BN = min(65536, NNZ // 8)
    J = NNZ // (8 * BN)
    H = W1.shape[0]

    vals4 = A_values.reshape(R, J, 8, BN)

    def body(*refs):
        _fused_body(float(NNZ), R, J, *refs)

    zero4 = lambda p, r, j: (0, 0, 0, 0)
    vout4, att_c = pl.pallas_call(
        body,
        grid=(2, R, J),
        in_specs=[
            pl.BlockSpec((1, 1, 8, BN),
                         lambda p, r, j: (r * (1 - p), j * (1 - p), 0, 0)),
            pl.BlockSpec((R, 1), lambda p, r, j: (0, 0)),
            pl.BlockSpec((R, 1), lambda p, r, j: (0, 0)),
            pl.BlockSpec((R, H), lambda p, r, j: (0, 0)),
            pl.BlockSpec((1, H), lambda p, r, j: (0, 0)),
            pl.BlockSpec((R, H), lambda p, r, j: (0, 0)),
            pl.BlockSpec((R, 1), lambda p, r, j: (0, 0)),
        ],
        out_specs=[
            pl.BlockSpec((1, 1, 8, BN), lambda p, r, j: (r * p, j * p, 0, 0)),
            pl.BlockSpec((R, 1), lambda p, r, j: (0, 0)),
        ],
        out_shape=[
            jax.ShapeDtypeStruct((R, J, 8, BN), jnp.float32),
            jax.ShapeDtypeStruct((R, 1), jnp.float32),
        ],
        scratch_shapes=[
            pltpu.VMEM((R * J, 8, BN), jnp.float32),
            pltpu.VMEM((R, BN), jnp.float32),
            pltpu.VMEM((R, 128), jnp.float32),
            pltpu.VMEM((R, 128), jnp.float32),
        ],
    )(vals4, rt_w.reshape(R, 1), rt_b.reshape(R, 1), W1.T,
      b1.reshape(1, H), W2, b2.reshape(R, 1))

    idx_out = _make_idx_copy(R, NNZ)(A_indices)

    return (idx_out.reshape(2, R * NNZ), vout4.reshape(R * NNZ),
            att_c.reshape(R))


# speedup vs baseline: 14.9643x; 14.9643x over previous
import jax
import jax.numpy as jnp
from jax.experimental import pallas as pl
from jax.experimental.pallas import tpu as pltpu


def _reduce_mlp_body(nnzf, vals_ref, rtw_ref, rtb_ref, w1t_ref, b1_ref,
                     w2_ref, b2_ref, att_ref, sc_ref, of_ref, acc_ref):
    j = pl.program_id(0)
    nb = pl.num_programs(0)

    @pl.when(j == 0)
    def _():
        acc_ref[...] = jnp.zeros_like(acc_ref)

    acc_ref[...] += vals_ref[...]

    @pl.when(j == nb - 1)
    def _():
        sums = jnp.sum(acc_ref[...], axis=1, keepdims=True)
        feats = rtw_ref[...] * sums + nnzf * rtb_ref[...]
        h = jnp.sum(feats * w1t_ref[...], axis=0, keepdims=True) + b1_ref[...]
        h = jnp.maximum(h, 0.0)
        logits = (jnp.sum(h * w2_ref[...], axis=1, keepdims=True)
                  + b2_ref[...])
        m = jnp.max(logits, axis=0, keepdims=True)
        e = jnp.exp(logits - m)
        att = e / jnp.sum(e, axis=0, keepdims=True)
        att_ref[...] = att
        sc_ref[...] = att * rtw_ref[...]
        of_ref[...] = att * rtb_ref[...]


def _scale_body(vals_ref, sc_ref, of_ref, vout_ref):
    vout_ref[...] = vals_ref[...] * sc_ref[...] + of_ref[...]


def kernel(A_indices, A_values, rt_w, rt_b, W1, b1, W2, b2):
    R, NNZ = A_values.shape
    BN = min(65536, NNZ)
    nb = NNZ // BN
    H = W1.shape[0]

    def body(*refs):
        _reduce_mlp_body(float(NNZ), *refs)

    small = lambda j: (0, 0)
    att_c, scale_c, off_c = pl.pallas_call(
        body,
        grid=(nb,),
        in_specs=[
            pl.BlockSpec((R, BN), lambda j: (0, j)),
            pl.BlockSpec((R, 1), small),
            pl.BlockSpec((R, 1), small),
            pl.BlockSpec((R, H), small),
            pl.BlockSpec((1, H), small),
            pl.BlockSpec((R, H), small),
            pl.BlockSpec((R, 1), small),
        ],
        out_specs=[pl.BlockSpec((R, 1), small)] * 3,
        out_shape=[jax.ShapeDtypeStruct((R, 1), jnp.float32)] * 3,
        scratch_shapes=[pltpu.VMEM((R, BN), jnp.float32)],
    )(A_values, rt_w.reshape(R, 1), rt_b.reshape(R, 1), W1.T,
      b1.reshape(1, H), W2, b2.reshape(R, 1))

    vals_out = pl.pallas_call(
        _scale_body,
        grid=(nb,),
        in_specs=[
            pl.BlockSpec((R, BN), lambda j: (0, j)),
            pl.BlockSpec((R, 1), small),
            pl.BlockSpec((R, 1), small),
        ],
        out_specs=pl.BlockSpec((R, BN), lambda j: (0, j)),
        out_shape=jax.ShapeDtypeStruct((R, NNZ), jnp.float32),
    )(A_values, scale_c, off_c)

    idx_out = jnp.transpose(A_indices, (1, 0, 2))

    return (idx_out.reshape(2, R * NNZ), vals_out.reshape(R * NNZ),
            att_c.reshape(R))


# speedup vs baseline: 15.0373x; 1.0049x over previous
import jax
import jax.numpy as jnp
from jax.experimental import pallas as pl
from jax.experimental.pallas import tpu as pltpu


def _reduce_mlp_body(nnzf, vals_ref, rtw_ref, rtb_ref, w1t_ref, b1_ref,
                     w2_ref, b2_ref, att_ref, sc_ref, of_ref, acc_ref):
    j = pl.program_id(0)
    nb = pl.num_programs(0)

    @pl.when(j == 0)
    def _():
        acc_ref[...] = jnp.zeros_like(acc_ref)

    acc_ref[...] += vals_ref[...]

    @pl.when(j == nb - 1)
    def _():
        sums = jnp.sum(acc_ref[...], axis=1, keepdims=True)
        feats = rtw_ref[...] * sums + nnzf * rtb_ref[...]
        h = jnp.sum(feats * w1t_ref[...], axis=0, keepdims=True) + b1_ref[...]
        h = jnp.maximum(h, 0.0)
        logits = (jnp.sum(h * w2_ref[...], axis=1, keepdims=True)
                  + b2_ref[...])
        m = jnp.max(logits, axis=0, keepdims=True)
        e = jnp.exp(logits - m)
        att = e / jnp.sum(e, axis=0, keepdims=True)
        att_ref[...] = att
        sc_ref[...] = att * rtw_ref[...]
        of_ref[...] = att * rtb_ref[...]


def _scale_body(vals_ref, sc_ref, of_ref, vout_ref):
    vout_ref[...] = vals_ref[...] * sc_ref[...] + of_ref[...]


def kernel(A_indices, A_values, rt_w, rt_b, W1, b1, W2, b2):
    R, NNZ = A_values.shape
    BN = min(65536, NNZ)
    nb = NNZ // BN
    H = W1.shape[0]

    idx_out = jnp.transpose(A_indices, (1, 0, 2))

    def body(*refs):
        _reduce_mlp_body(float(NNZ), *refs)

    small = lambda j: (0, 0)
    att_c, scale_c, off_c = pl.pallas_call(
        body,
        grid=(nb,),
        in_specs=[
            pl.BlockSpec((R, BN), lambda j: (0, j)),
            pl.BlockSpec((R, 1), small),
            pl.BlockSpec((R, 1), small),
            pl.BlockSpec((R, H), small),
            pl.BlockSpec((1, H), small),
            pl.BlockSpec((R, H), small),
            pl.BlockSpec((R, 1), small),
        ],
        out_specs=[pl.BlockSpec((R, 1), small)] * 3,
        out_shape=[jax.ShapeDtypeStruct((R, 1), jnp.float32)] * 3,
        scratch_shapes=[pltpu.VMEM((R, BN), jnp.float32)],
    )(A_values, rt_w.reshape(R, 1), rt_b.reshape(R, 1), W1.T,
      b1.reshape(1, H), W2, b2.reshape(R, 1))

    vals_out = pl.pallas_call(
        _scale_body,
        grid=(nb,),
        in_specs=[
            pl.BlockSpec((R, BN), lambda j: (0, j)),
            pl.BlockSpec((R, 1), small),
            pl.BlockSpec((R, 1), small),
        ],
        out_specs=pl.BlockSpec((R, BN), lambda j: (0, j)),
        out_shape=jax.ShapeDtypeStruct((R, NNZ), jnp.float32),
    )(A_values, scale_c, off_c)

    return (idx_out.reshape(2, R * NNZ), vals_out.reshape(R * NNZ),
            att_c.reshape(R))
